# baseline (speedup 1.0000x reference)
import functools

import jax
import jax.numpy as jnp
from jax import lax
from jax.experimental import pallas as pl
from jax.experimental.pallas import tpu as pltpu
from jax.experimental.pallas import tpu_sc as plsc

NC, NS = 2, 16
NW = NC * NS
CHUNK = 80

_SC_MESH = dict(core_axis_name="c", subcore_axis_name="s", num_cores=NC,
                num_subcores=NS)


def _node_pre_body(h_ref, g_ref, b_ref, ws_ref, wd_ref, be1_ref,
                   hn_ref, a_ref, bb_ref):
    h = h_ref[...]
    mu = jnp.mean(h, axis=1, keepdims=True)
    var = jnp.mean((h - mu) ** 2, axis=1, keepdims=True)
    hn = (h - mu) * lax.rsqrt(var + 1e-5) * g_ref[...] + b_ref[...]
    hn_ref[...] = hn
    a_ref[...] = jnp.dot(hn, ws_ref[...], preferred_element_type=jnp.float32)
    bb_ref[...] = (jnp.dot(hn, wd_ref[...], preferred_element_type=jnp.float32)
                   + be1_ref[...])


def _node_pre(h, ln_g, ln_b, w_src, w_dst, be1, tn):
    n, d = h.shape
    grid = n // tn
    blk = lambda i: (i, 0)
    full = lambda i: (0, 0)
    out = jax.ShapeDtypeStruct((n, d), jnp.float32)
    return pl.pallas_call(
        _node_pre_body,
        grid=(grid,),
        in_specs=[pl.BlockSpec((tn, d), blk),
                  pl.BlockSpec((1, d), full), pl.BlockSpec((1, d), full),
                  pl.BlockSpec((d, d), full), pl.BlockSpec((d, d), full),
                  pl.BlockSpec((1, d), full)],
        out_specs=[pl.BlockSpec((tn, d), blk)] * 3,
        out_shape=[out, out, out],
    )(h, ln_g.reshape(1, d), ln_b.reshape(1, d), w_src, w_dst,
      be1.reshape(1, d))


GC = 40


def _add_pack(ra, rb, rc, rd, pk, d):
    msk = jnp.int32(-65536)
    half = d // 2

    def pk16(lo_f, hi_f):
        ulo = lax.shift_right_logical(
            lax.bitcast_convert_type(lo_f, jnp.int32) + 0x8000, 16)
        uhi = (lax.bitcast_convert_type(hi_f, jnp.int32) + 0x8000) & msk
        return uhi | ulo

    def row(i, _):
        for j in range(half // 16):
            sl = pl.ds(j * 16, 16)
            sh = pl.ds(half + j * 16, 16)
            pk[i, sl] = pk16(ra[i, sl] + rb[i, sl], ra[i, sh] + rb[i, sh])
            pk[i, sh] = pk16(rc[i, sl] + rd[i, sl], rc[i, sh] + rd[i, sh])
        return 0

    lax.fori_loop(0, GC, row, 0)


def _gather_body(a_hbm, b_hbm, src_hbm, dst_hbm, og_hbm,
                 si0_v, di0_v, si1_v, di1_v,
                 ra0, rb0, rc0, rd0, pk0, ra1, rb1, rc1, rd1, pk1,
                 s1, s2, s3, s4, s5, s6, s7, s8, sw0, sw1):
    e = src_hbm.shape[0]
    d = a_hbm.shape[1]
    eh = e // 2
    epw = eh // NW
    wid = lax.axis_index("s") * NC + lax.axis_index("c")
    base0 = wid * epw

    pltpu.sync_copy(src_hbm.at[pl.ds(base0, epw)], si0_v)
    pltpu.sync_copy(dst_hbm.at[pl.ds(base0, epw)], di0_v)
    pltpu.sync_copy(src_hbm.at[pl.ds(eh + base0, epw)], si1_v)
    pltpu.sync_copy(dst_hbm.at[pl.ds(eh + base0, epw)], di1_v)

    nch = epw // GC

    def chunk(k, bufs):
        ra, rb, rc, rd, pk, sa, sb, sc, sd, sw = bufs
        q = pl.ds(k * GC, GC)
        cpa = pltpu.async_copy(a_hbm.at[si0_v.at[q]], ra, sa)
        cpb = pltpu.async_copy(b_hbm.at[di0_v.at[q]], rb, sb)
        cpc = pltpu.async_copy(a_hbm.at[si1_v.at[q]], rc, sc)
        cpd = pltpu.async_copy(b_hbm.at[di1_v.at[q]], rd, sd)
        cpa.wait()
        cpb.wait()
        cpc.wait()
        cpd.wait()
        _add_pack(ra, rb, rc, rd, pk, d)
        return pltpu.async_copy(pk, og_hbm.at[pl.ds(base0 + k * GC, GC)], sw)

    bufs0 = (ra0, rb0, rc0, rd0, pk0, s1, s2, s3, s4, sw0)
    bufs1 = (ra1, rb1, rc1, rd1, pk1, s5, s6, s7, s8, sw1)

    def pair(kk, _):
        w0 = chunk(kk * 2, bufs0)
        w1 = chunk(kk * 2 + 1, bufs1)
        w0.wait()
        w1.wait()
        return 0

    lax.fori_loop(0, nch // 2, pair, 0)
    chunk(nch - 1, bufs0).wait()


def _gather(a, b, src, dst):
    n, d = a.shape
    e = src.shape[0]
    epw = e // 2 // NW
    row_f32 = pltpu.VMEM((GC, d), jnp.float32)
    row_i32 = pltpu.VMEM((GC, d), jnp.int32)
    fn = pl.kernel(
        _gather_body,
        out_type=jax.ShapeDtypeStruct((e // 2, d), jnp.int32),
        mesh=plsc.VectorSubcoreMesh(**_SC_MESH),
        scratch_types=[
            pltpu.VMEM((epw,), jnp.int32),
            pltpu.VMEM((epw,), jnp.int32),
            pltpu.VMEM((epw,), jnp.int32),
            pltpu.VMEM((epw,), jnp.int32),
            row_f32, row_f32, row_f32, row_f32, row_i32,
            row_f32, row_f32, row_f32, row_f32, row_i32,
        ] + [pltpu.SemaphoreType.DMA] * 10,
    )
    return fn(a, b, src, dst)


def _edge_half(gp, gram, rbf, wg_ref, wr_ref, we2lo_ref, we2hi_ref, be2_ref):
    hd = gp.shape[1]
    xlo = lax.bitcast_convert_type(lax.shift_left(gp, 16), jnp.float32)
    xhi = lax.bitcast_convert_type(gp & jnp.int32(-65536), jnp.float32)
    c = jnp.dot(gram, wg_ref[...], preferred_element_type=jnp.float32)
    c = c + jnp.dot(rbf, wr_ref[...], preferred_element_type=jnp.float32)
    xlo = xlo + c[:, :hd]
    xhi = xhi + c[:, hd:]
    m1lo = xlo * jax.nn.sigmoid(xlo)
    m1hi = xhi * jax.nn.sigmoid(xhi)
    y = (jnp.dot(m1lo, we2lo_ref[...], preferred_element_type=jnp.float32)
         + jnp.dot(m1hi, we2hi_ref[...], preferred_element_type=jnp.float32)
         + be2_ref[...])
    return y * jax.nn.sigmoid(y)


def _edge_body(g_ref, gram0_ref, rbf0_ref, gram1_ref, rbf1_ref,
               wg_ref, wr_ref, we2lo_ref, we2hi_ref, be2_ref, m2_ref):
    gp = g_ref[...]
    hd = gp.shape[1] // 2
    m2_ref[0] = _edge_half(gp[:, :hd], gram0_ref[...], rbf0_ref[...],
                           wg_ref, wr_ref, we2lo_ref, we2hi_ref, be2_ref)
    m2_ref[1] = _edge_half(gp[:, hd:], gram1_ref[...], rbf1_ref[...],
                           wg_ref, wr_ref, we2lo_ref, we2hi_ref, be2_ref)


def _edge_mlp(gsum, gram, rbf, w_g, w_r, we2, be2, te):
    eh, d = gsum.shape
    hd = d // 2
    g = gram.shape[1]
    k = rbf.shape[1]
    grid = eh // te
    off = eh // te
    blk = lambda i: (i, 0)
    blk1 = lambda i: (i + off, 0)
    blk3 = lambda i: (0, i, 0)
    full = lambda i: (0, 0)
    m2 = pl.pallas_call(
        _edge_body,
        grid=(grid,),
        in_specs=[pl.BlockSpec((te, d), blk),
                  pl.BlockSpec((te, g), blk), pl.BlockSpec((te, k), blk),
                  pl.BlockSpec((te, g), blk1), pl.BlockSpec((te, k), blk1),
                  pl.BlockSpec((g, d), full), pl.BlockSpec((k, d), full),
                  pl.BlockSpec((hd, d), full), pl.BlockSpec((hd, d), full),
                  pl.BlockSpec((1, d), full)],
        out_specs=pl.BlockSpec((2, te, d), blk3),
        out_shape=jax.ShapeDtypeStruct((2, eh, d), jnp.float32),
    )(gsum, gram, rbf, gram, rbf, w_g, w_r, we2[:hd], we2[hd:],
      be2.reshape(1, d))
    return m2.reshape(2 * eh, d)


def _acc_init(zn_hbm, acc_s, stage_v, s, n):
    slab = n - (NS - 1) * 624
    nst = slab // 40

    def init(j, _):
        r = pl.ds(s * 624 + j * 40, 40)
        pltpu.sync_copy(zn_hbm.at[r], stage_v)
        pltpu.sync_copy(stage_v, acc_s.at[r])
        return 0

    lax.fori_loop(0, nst, init, 0)


def _acc_dump(acc_s, out_hbm, stage_v, c, s, n):
    slab = n - (NS - 1) * 624
    nst = slab // 40

    def dump(j, _):
        r = pl.ds(s * 624 + j * 40, 40)
        pltpu.sync_copy(acc_s.at[r], stage_v)
        pltpu.sync_copy(stage_v, out_hbm.at[c, r])
        return 0

    lax.fori_loop(0, nst, dump, 0)


def _scatter_sums_body(m2_hbm, dst_hbm, zn_hbm,
                       sums_hbm,
                       m20_v, m21_v, di0_v, di1_v, stage_v, acc_s,
                       sm0, sm1, si0, si1, sc0, sc1):
    e = dst_hbm.shape[0]
    n = zn_hbm.shape[0]
    epw = e // NW
    c = lax.axis_index("c")
    s = lax.axis_index("s")
    wid = s * NC + c
    _acc_init(zn_hbm, acc_s, stage_v, s, n)
    plsc.subcore_barrier()

    base0 = wid * epw
    nch = epw // CHUNK

    def pair(kk, _):
        base = base0 + kk * 2 * CHUNK
        ci0 = pltpu.async_copy(dst_hbm.at[pl.ds(base, CHUNK)], di0_v, si0)
        ci1 = pltpu.async_copy(
            dst_hbm.at[pl.ds(base + CHUNK, CHUNK)], di1_v, si1)
        cm0 = pltpu.async_copy(m2_hbm.at[pl.ds(base, CHUNK)], m20_v, sm0)
        cm1 = pltpu.async_copy(
            m2_hbm.at[pl.ds(base + CHUNK, CHUNK)], m21_v, sm1)
        ci0.wait()
        cm0.wait()
        a0 = pltpu.async_copy(m20_v, acc_s.at[di0_v], sc0, add=True)
        ci1.wait()
        cm1.wait()
        a1 = pltpu.async_copy(m21_v, acc_s.at[di1_v], sc1, add=True)
        a0.wait()
        a1.wait()
        return 0

    lax.fori_loop(0, nch // 2, pair, 0)

    base = base0 + (nch - 1) * CHUNK
    pltpu.sync_copy(dst_hbm.at[pl.ds(base, CHUNK)], di0_v)
    pltpu.sync_copy(m2_hbm.at[pl.ds(base, CHUNK)], m20_v)
    pltpu.sync_copy(m20_v, acc_s.at[di0_v], add=True)

    plsc.subcore_barrier()
    _acc_dump(acc_s, sums_hbm, stage_v, c, s, n)


def _scatter_counts_body(dst_hbm, zn_hbm, ones_hbm,
                         cnt_hbm,
                         ones_v, di0_v, di1_v, stage_v, acc_s,
                         si0, si1, sc0, sc1):
    e = dst_hbm.shape[0]
    n = zn_hbm.shape[0]
    epw = e // NW
    c = lax.axis_index("c")
    s = lax.axis_index("s")
    wid = s * NC + c
    _acc_init(zn_hbm, acc_s, stage_v, s, n)
    pltpu.sync_copy(ones_hbm, ones_v)
    plsc.subcore_barrier()

    base0 = wid * epw
    nch = epw // CHUNK

    def pair(kk, _):
        base = base0 + kk * 2 * CHUNK
        ci0 = pltpu.async_copy(dst_hbm.at[pl.ds(base, CHUNK)], di0_v, si0)
        ci1 = pltpu.async_copy(
            dst_hbm.at[pl.ds(base + CHUNK, CHUNK)], di1_v, si1)
        ci0.wait()
        a0 = pltpu.async_copy(ones_v, acc_s.at[di0_v], sc0, add=True)
        ci1.wait()
        a1 = pltpu.async_copy(ones_v, acc_s.at[di1_v], sc1, add=True)
        a0.wait()
        a1.wait()
        return 0

    lax.fori_loop(0, nch // 2, pair, 0)

    base = base0 + (nch - 1) * CHUNK
    pltpu.sync_copy(dst_hbm.at[pl.ds(base, CHUNK)], di0_v)
    pltpu.sync_copy(ones_v, acc_s.at[di0_v], add=True)

    plsc.subcore_barrier()
    _acc_dump(acc_s, cnt_hbm, stage_v, c, s, n)


def _scatter_sums(m2, dst, n):
    e, d = m2.shape
    zn = jnp.zeros((n, d), jnp.float32)
    return pl.kernel(
        _scatter_sums_body,
        out_type=jax.ShapeDtypeStruct((NC, n, d), jnp.float32),
        mesh=plsc.VectorSubcoreMesh(**_SC_MESH),
        scratch_types=[
            pltpu.VMEM((CHUNK, d), jnp.float32),
            pltpu.VMEM((CHUNK, d), jnp.float32),
            pltpu.VMEM((CHUNK,), jnp.int32),
            pltpu.VMEM((CHUNK,), jnp.int32),
            pltpu.VMEM((40, d), jnp.float32),
            pltpu.VMEM_SHARED((n, d), jnp.float32),
            pltpu.SemaphoreType.DMA,
            pltpu.SemaphoreType.DMA,
            pltpu.SemaphoreType.DMA,
            pltpu.SemaphoreType.DMA,
            pltpu.SemaphoreType.DMA,
            pltpu.SemaphoreType.DMA,
        ],
    )(m2, dst, zn)


def _scatter_counts(dst, n, d):
    zn = jnp.zeros((n, d), jnp.float32)
    ones = jnp.ones((CHUNK, d), jnp.float32)
    return pl.kernel(
        _scatter_counts_body,
        out_type=jax.ShapeDtypeStruct((NC, n, d), jnp.float32),
        mesh=plsc.VectorSubcoreMesh(**_SC_MESH),
        scratch_types=[
            pltpu.VMEM((CHUNK, d), jnp.float32),
            pltpu.VMEM((CHUNK,), jnp.int32),
            pltpu.VMEM((CHUNK,), jnp.int32),
            pltpu.VMEM((40, d), jnp.float32),
            pltpu.VMEM_SHARED((n, d), jnp.float32),
            pltpu.SemaphoreType.DMA,
            pltpu.SemaphoreType.DMA,
            pltpu.SemaphoreType.DMA,
            pltpu.SemaphoreType.DMA,
        ],
    )(dst, zn, ones)


def _node_upd_body(h_ref, hn_ref, sums_ref, cnt_ref, w1h_ref, w1m_ref,
                   bn1_ref, wn2_ref, bn2_ref, out_ref):
    cnt = cnt_ref[0, :, 0:1] + cnt_ref[1, :, 0:1]
    m = (sums_ref[0] + sums_ref[1]) / jnp.maximum(cnt, 1.0)
    t = (jnp.dot(hn_ref[...], w1h_ref[...], preferred_element_type=jnp.float32)
         + jnp.dot(m, w1m_ref[...], preferred_element_type=jnp.float32)
         + bn1_ref[...])
    t = t * jax.nn.sigmoid(t)
    y = jnp.dot(t, wn2_ref[...], preferred_element_type=jnp.float32) \
        + bn2_ref[...]
    out_ref[...] = h_ref[...] + y * jax.nn.sigmoid(y)


def _node_update(h, hn, sums, cnt, w1h, w1m, bn1, wn2, bn2, tn):
    n, d = h.shape
    grid = n // tn
    blk = lambda i: (i, 0)
    blk3 = lambda i: (0, i, 0)
    full = lambda i: (0, 0)
    return pl.pallas_call(
        _node_upd_body,
        grid=(grid,),
        in_specs=[pl.BlockSpec((tn, d), blk), pl.BlockSpec((tn, d), blk),
                  pl.BlockSpec((NC, tn, d), blk3),
                  pl.BlockSpec((NC, tn, d), blk3),
                  pl.BlockSpec((d, d), full), pl.BlockSpec((d, d), full),
                  pl.BlockSpec((1, d), full), pl.BlockSpec((d, d), full),
                  pl.BlockSpec((1, d), full)],
        out_specs=pl.BlockSpec((tn, d), blk),
        out_shape=jax.ShapeDtypeStruct((n, d), jnp.float32),
    )(h, hn, sums, cnt, w1h, w1m, bn1.reshape(1, d), wn2, bn2.reshape(1, d))


def kernel(h, rbf_edge, gram_edge, edge_index, ln_g, ln_b,
           We1, be1, We2, be2, Wn1, bn1, Wn2, bn2):
    n, d = h.shape
    src = edge_index[0]
    dst = edge_index[1]
    g = gram_edge.shape[1]

    w_src = We1[:d]
    w_dst = We1[d:2 * d]
    w_g = We1[2 * d:2 * d + g]
    w_r = We1[2 * d + g:]

    hn, a_proj, b_proj = _node_pre(h, ln_g, ln_b, w_src, w_dst, be1, tn=1000)
    gsum = _gather(a_proj, b_proj, src, dst)
    cnt = _scatter_counts(dst, n, d)
    m2 = _edge_mlp(gsum, gram_edge, rbf_edge, w_g, w_r, We2, be2,
                   te=2000)
    sums = _scatter_sums(m2, dst, n)
    return _node_update(h, hn, sums, cnt, Wn1[:d], Wn1[d:], bn1, Wn2, bn2,
                        tn=1000)


# speedup vs baseline: 1.0689x; 1.0689x over previous
import functools

import jax
import jax.numpy as jnp
from jax import lax
from jax.experimental import pallas as pl
from jax.experimental.pallas import tpu as pltpu
from jax.experimental.pallas import tpu_sc as plsc

NC, NS = 2, 16
NW = NC * NS
CHUNK = 80

_SC_MESH = dict(core_axis_name="c", subcore_axis_name="s", num_cores=NC,
                num_subcores=NS)


def _node_pre_body(h_ref, g_ref, b_ref, ws_ref, wd_ref, be1_ref,
                   hn_ref, a_ref, bb_ref):
    h = h_ref[...]
    mu = jnp.mean(h, axis=1, keepdims=True)
    var = jnp.mean((h - mu) ** 2, axis=1, keepdims=True)
    hn = (h - mu) * lax.rsqrt(var + 1e-5) * g_ref[...] + b_ref[...]
    hn_ref[...] = hn
    a_ref[...] = jnp.dot(hn, ws_ref[...], preferred_element_type=jnp.float32)
    bb_ref[...] = (jnp.dot(hn, wd_ref[...], preferred_element_type=jnp.float32)
                   + be1_ref[...])


def _node_pre(h, ln_g, ln_b, w_src, w_dst, be1, tn):
    n, d = h.shape
    grid = n // tn
    blk = lambda i: (i, 0)
    full = lambda i: (0, 0)
    out = jax.ShapeDtypeStruct((n, d), jnp.float32)
    return pl.pallas_call(
        _node_pre_body,
        grid=(grid,),
        in_specs=[pl.BlockSpec((tn, d), blk),
                  pl.BlockSpec((1, d), full), pl.BlockSpec((1, d), full),
                  pl.BlockSpec((d, d), full), pl.BlockSpec((d, d), full),
                  pl.BlockSpec((1, d), full)],
        out_specs=[pl.BlockSpec((tn, d), blk)] * 3,
        out_shape=[out, out, out],
    )(h, ln_g.reshape(1, d), ln_b.reshape(1, d), w_src, w_dst,
      be1.reshape(1, d))


GC = 40


def _add_pack(ra, rb, rc, rd, pk, d):
    msk = jnp.int32(-65536)
    half = d // 2

    def pk16(lo_f, hi_f):
        ulo = lax.shift_right_logical(
            lax.bitcast_convert_type(lo_f, jnp.int32) + 0x8000, 16)
        uhi = (lax.bitcast_convert_type(hi_f, jnp.int32) + 0x8000) & msk
        return uhi | ulo

    def row(i, _):
        for j in range(half // 16):
            sl = pl.ds(j * 16, 16)
            sh = pl.ds(half + j * 16, 16)
            pk[i, sl] = pk16(ra[i, sl] + rb[i, sl], ra[i, sh] + rb[i, sh])
            pk[i, sh] = pk16(rc[i, sl] + rd[i, sl], rc[i, sh] + rd[i, sh])
        return 0

    lax.fori_loop(0, GC, row, 0)


def _gather_body(a_hbm, b_hbm, src_hbm, dst_hbm, og_hbm,
                 si0_v, di0_v, si1_v, di1_v,
                 ra0, rb0, rc0, rd0, pk0, ra1, rb1, rc1, rd1, pk1,
                 s1, s2, s3, s4, s5, s6, s7, s8, sw0, sw1):
    e = src_hbm.shape[0]
    d = a_hbm.shape[1]
    eh = e // 2
    epw = eh // NW
    wid = lax.axis_index("s") * NC + lax.axis_index("c")
    base0 = wid * epw

    pltpu.sync_copy(src_hbm.at[pl.ds(base0, epw)], si0_v)
    pltpu.sync_copy(dst_hbm.at[pl.ds(base0, epw)], di0_v)
    pltpu.sync_copy(src_hbm.at[pl.ds(eh + base0, epw)], si1_v)
    pltpu.sync_copy(dst_hbm.at[pl.ds(eh + base0, epw)], di1_v)

    nch = epw // GC

    def issue(k, ra, rb, rc, rd, sa, sb, sc, sd):
        q = pl.ds(k * GC, GC)
        return (pltpu.async_copy(a_hbm.at[si0_v.at[q]], ra, sa),
                pltpu.async_copy(b_hbm.at[di0_v.at[q]], rb, sb),
                pltpu.async_copy(a_hbm.at[si1_v.at[q]], rc, sc),
                pltpu.async_copy(b_hbm.at[di1_v.at[q]], rd, sd))

    def pair(kk, _):
        ka = kk * 2
        kb = ka + 1
        cps0 = issue(ka, ra0, rb0, rc0, rd0, s1, s2, s3, s4)
        cps1 = issue(kb, ra1, rb1, rc1, rd1, s5, s6, s7, s8)
        for cp in cps0:
            cp.wait()
        _add_pack(ra0, rb0, rc0, rd0, pk0, d)
        w0 = pltpu.async_copy(pk0, og_hbm.at[pl.ds(base0 + ka * GC, GC)],
                              sw0)
        for cp in cps1:
            cp.wait()
        _add_pack(ra1, rb1, rc1, rd1, pk1, d)
        w1 = pltpu.async_copy(pk1, og_hbm.at[pl.ds(base0 + kb * GC, GC)],
                              sw1)
        w0.wait()
        w1.wait()
        return 0

    lax.fori_loop(0, nch // 2, pair, 0)

    cps0 = issue(nch - 1, ra0, rb0, rc0, rd0, s1, s2, s3, s4)
    for cp in cps0:
        cp.wait()
    _add_pack(ra0, rb0, rc0, rd0, pk0, d)
    pltpu.async_copy(pk0, og_hbm.at[pl.ds(base0 + (nch - 1) * GC, GC)],
                     sw0).wait()


def _gather(a, b, src, dst):
    n, d = a.shape
    e = src.shape[0]
    epw = e // 2 // NW
    row_f32 = pltpu.VMEM((GC, d), jnp.float32)
    row_i32 = pltpu.VMEM((GC, d), jnp.int32)
    fn = pl.kernel(
        _gather_body,
        out_type=jax.ShapeDtypeStruct((e // 2, d), jnp.int32),
        mesh=plsc.VectorSubcoreMesh(**_SC_MESH),
        scratch_types=[
            pltpu.VMEM((epw,), jnp.int32),
            pltpu.VMEM((epw,), jnp.int32),
            pltpu.VMEM((epw,), jnp.int32),
            pltpu.VMEM((epw,), jnp.int32),
            row_f32, row_f32, row_f32, row_f32, row_i32,
            row_f32, row_f32, row_f32, row_f32, row_i32,
        ] + [pltpu.SemaphoreType.DMA] * 10,
    )
    return fn(a, b, src, dst)


def _edge_half(gp, gram, rbf, wg_ref, wr_ref, we2lo_ref, we2hi_ref, be2_ref):
    hd = gp.shape[1]
    xlo = lax.bitcast_convert_type(lax.shift_left(gp, 16), jnp.float32)
    xhi = lax.bitcast_convert_type(gp & jnp.int32(-65536), jnp.float32)
    c = jnp.dot(gram, wg_ref[...], preferred_element_type=jnp.float32)
    c = c + jnp.dot(rbf, wr_ref[...], preferred_element_type=jnp.float32)
    xlo = xlo + c[:, :hd]
    xhi = xhi + c[:, hd:]
    m1lo = xlo * jax.nn.sigmoid(xlo)
    m1hi = xhi * jax.nn.sigmoid(xhi)
    y = (jnp.dot(m1lo, we2lo_ref[...], preferred_element_type=jnp.float32)
         + jnp.dot(m1hi, we2hi_ref[...], preferred_element_type=jnp.float32)
         + be2_ref[...])
    return y * jax.nn.sigmoid(y)


def _edge_body(g_ref, gram0_ref, rbf0_ref, gram1_ref, rbf1_ref,
               wg_ref, wr_ref, we2lo_ref, we2hi_ref, be2_ref, m2_ref):
    gp = g_ref[...]
    hd = gp.shape[1] // 2
    m2_ref[0] = _edge_half(gp[:, :hd], gram0_ref[...], rbf0_ref[...],
                           wg_ref, wr_ref, we2lo_ref, we2hi_ref, be2_ref)
    m2_ref[1] = _edge_half(gp[:, hd:], gram1_ref[...], rbf1_ref[...],
                           wg_ref, wr_ref, we2lo_ref, we2hi_ref, be2_ref)


def _edge_mlp(gsum, gram, rbf, w_g, w_r, we2, be2, te):
    eh, d = gsum.shape
    hd = d // 2
    g = gram.shape[1]
    k = rbf.shape[1]
    grid = eh // te
    off = eh // te
    blk = lambda i: (i, 0)
    blk1 = lambda i: (i + off, 0)
    blk3 = lambda i: (0, i, 0)
    full = lambda i: (0, 0)
    m2 = pl.pallas_call(
        _edge_body,
        grid=(grid,),
        in_specs=[pl.BlockSpec((te, d), blk),
                  pl.BlockSpec((te, g), blk), pl.BlockSpec((te, k), blk),
                  pl.BlockSpec((te, g), blk1), pl.BlockSpec((te, k), blk1),
                  pl.BlockSpec((g, d), full), pl.BlockSpec((k, d), full),
                  pl.BlockSpec((hd, d), full), pl.BlockSpec((hd, d), full),
                  pl.BlockSpec((1, d), full)],
        out_specs=pl.BlockSpec((2, te, d), blk3),
        out_shape=jax.ShapeDtypeStruct((2, eh, d), jnp.float32),
    )(gsum, gram, rbf, gram, rbf, w_g, w_r, we2[:hd], we2[hd:],
      be2.reshape(1, d))
    return m2.reshape(2 * eh, d)


def _acc_init(zn_hbm, acc_s, stage_v, s, n):
    slab = n - (NS - 1) * 624
    nst = slab // 40

    def init(j, _):
        r = pl.ds(s * 624 + j * 40, 40)
        pltpu.sync_copy(zn_hbm.at[r], stage_v)
        pltpu.sync_copy(stage_v, acc_s.at[r])
        return 0

    lax.fori_loop(0, nst, init, 0)


def _acc_dump(acc_s, out_hbm, stage_v, c, s, n):
    slab = n - (NS - 1) * 624
    nst = slab // 40

    def dump(j, _):
        r = pl.ds(s * 624 + j * 40, 40)
        pltpu.sync_copy(acc_s.at[r], stage_v)
        pltpu.sync_copy(stage_v, out_hbm.at[c, r])
        return 0

    lax.fori_loop(0, nst, dump, 0)


def _scatter_sums_body(m2_hbm, dst_hbm, zn_hbm,
                       sums_hbm,
                       m20_v, m21_v, di0_v, di1_v, stage_v, acc_s,
                       sm0, sm1, si0, si1, sc0, sc1):
    e = dst_hbm.shape[0]
    n = zn_hbm.shape[0]
    epw = e // NW
    c = lax.axis_index("c")
    s = lax.axis_index("s")
    wid = s * NC + c
    _acc_init(zn_hbm, acc_s, stage_v, s, n)
    plsc.subcore_barrier()

    base0 = wid * epw
    nch = epw // CHUNK

    def pair(kk, _):
        base = base0 + kk * 2 * CHUNK
        ci0 = pltpu.async_copy(dst_hbm.at[pl.ds(base, CHUNK)], di0_v, si0)
        ci1 = pltpu.async_copy(
            dst_hbm.at[pl.ds(base + CHUNK, CHUNK)], di1_v, si1)
        cm0 = pltpu.async_copy(m2_hbm.at[pl.ds(base, CHUNK)], m20_v, sm0)
        cm1 = pltpu.async_copy(
            m2_hbm.at[pl.ds(base + CHUNK, CHUNK)], m21_v, sm1)
        ci0.wait()
        cm0.wait()
        a0 = pltpu.async_copy(m20_v, acc_s.at[di0_v], sc0, add=True)
        ci1.wait()
        cm1.wait()
        a1 = pltpu.async_copy(m21_v, acc_s.at[di1_v], sc1, add=True)
        a0.wait()
        a1.wait()
        return 0

    lax.fori_loop(0, nch // 2, pair, 0)

    base = base0 + (nch - 1) * CHUNK
    pltpu.sync_copy(dst_hbm.at[pl.ds(base, CHUNK)], di0_v)
    pltpu.sync_copy(m2_hbm.at[pl.ds(base, CHUNK)], m20_v)
    pltpu.sync_copy(m20_v, acc_s.at[di0_v], add=True)

    plsc.subcore_barrier()
    _acc_dump(acc_s, sums_hbm, stage_v, c, s, n)


def _scatter_counts_body(dst_hbm, zn_hbm, ones_hbm,
                         cnt_hbm,
                         ones_v, di0_v, di1_v, stage_v, acc_s,
                         si0, si1, sc0, sc1):
    e = dst_hbm.shape[0]
    n = zn_hbm.shape[0]
    epw = e // NW
    c = lax.axis_index("c")
    s = lax.axis_index("s")
    wid = s * NC + c
    _acc_init(zn_hbm, acc_s, stage_v, s, n)
    pltpu.sync_copy(ones_hbm, ones_v)
    plsc.subcore_barrier()

    base0 = wid * epw
    nch = epw // CHUNK

    def pair(kk, _):
        base = base0 + kk * 2 * CHUNK
        ci0 = pltpu.async_copy(dst_hbm.at[pl.ds(base, CHUNK)], di0_v, si0)
        ci1 = pltpu.async_copy(
            dst_hbm.at[pl.ds(base + CHUNK, CHUNK)], di1_v, si1)
        ci0.wait()
        a0 = pltpu.async_copy(ones_v, acc_s.at[di0_v], sc0, add=True)
        ci1.wait()
        a1 = pltpu.async_copy(ones_v, acc_s.at[di1_v], sc1, add=True)
        a0.wait()
        a1.wait()
        return 0

    lax.fori_loop(0, nch // 2, pair, 0)

    base = base0 + (nch - 1) * CHUNK
    pltpu.sync_copy(dst_hbm.at[pl.ds(base, CHUNK)], di0_v)
    pltpu.sync_copy(ones_v, acc_s.at[di0_v], add=True)

    plsc.subcore_barrier()
    _acc_dump(acc_s, cnt_hbm, stage_v, c, s, n)


def _scatter_sums(m2, dst, n):
    e, d = m2.shape
    zn = jnp.zeros((n, d), jnp.float32)
    return pl.kernel(
        _scatter_sums_body,
        out_type=jax.ShapeDtypeStruct((NC, n, d), jnp.float32),
        mesh=plsc.VectorSubcoreMesh(**_SC_MESH),
        scratch_types=[
            pltpu.VMEM((CHUNK, d), jnp.float32),
            pltpu.VMEM((CHUNK, d), jnp.float32),
            pltpu.VMEM((CHUNK,), jnp.int32),
            pltpu.VMEM((CHUNK,), jnp.int32),
            pltpu.VMEM((40, d), jnp.float32),
            pltpu.VMEM_SHARED((n, d), jnp.float32),
            pltpu.SemaphoreType.DMA,
            pltpu.SemaphoreType.DMA,
            pltpu.SemaphoreType.DMA,
            pltpu.SemaphoreType.DMA,
            pltpu.SemaphoreType.DMA,
            pltpu.SemaphoreType.DMA,
        ],
    )(m2, dst, zn)


def _scatter_counts(dst, n, d):
    zn = jnp.zeros((n, d), jnp.float32)
    ones = jnp.ones((CHUNK, d), jnp.float32)
    return pl.kernel(
        _scatter_counts_body,
        out_type=jax.ShapeDtypeStruct((NC, n, d), jnp.float32),
        mesh=plsc.VectorSubcoreMesh(**_SC_MESH),
        scratch_types=[
            pltpu.VMEM((CHUNK, d), jnp.float32),
            pltpu.VMEM((CHUNK,), jnp.int32),
            pltpu.VMEM((CHUNK,), jnp.int32),
            pltpu.VMEM((40, d), jnp.float32),
            pltpu.VMEM_SHARED((n, d), jnp.float32),
            pltpu.SemaphoreType.DMA,
            pltpu.SemaphoreType.DMA,
            pltpu.SemaphoreType.DMA,
            pltpu.SemaphoreType.DMA,
        ],
    )(dst, zn, ones)


def _node_upd_body(h_ref, hn_ref, sums_ref, cnt_ref, w1h_ref, w1m_ref,
                   bn1_ref, wn2_ref, bn2_ref, out_ref):
    cnt = cnt_ref[0, :, 0:1] + cnt_ref[1, :, 0:1]
    m = (sums_ref[0] + sums_ref[1]) / jnp.maximum(cnt, 1.0)
    t = (jnp.dot(hn_ref[...], w1h_ref[...], preferred_element_type=jnp.float32)
         + jnp.dot(m, w1m_ref[...], preferred_element_type=jnp.float32)
         + bn1_ref[...])
    t = t * jax.nn.sigmoid(t)
    y = jnp.dot(t, wn2_ref[...], preferred_element_type=jnp.float32) \
        + bn2_ref[...]
    out_ref[...] = h_ref[...] + y * jax.nn.sigmoid(y)


def _node_update(h, hn, sums, cnt, w1h, w1m, bn1, wn2, bn2, tn):
    n, d = h.shape
    grid = n // tn
    blk = lambda i: (i, 0)
    blk3 = lambda i: (0, i, 0)
    full = lambda i: (0, 0)
    return pl.pallas_call(
        _node_upd_body,
        grid=(grid,),
        in_specs=[pl.BlockSpec((tn, d), blk), pl.BlockSpec((tn, d), blk),
                  pl.BlockSpec((NC, tn, d), blk3),
                  pl.BlockSpec((NC, tn, d), blk3),
                  pl.BlockSpec((d, d), full), pl.BlockSpec((d, d), full),
                  pl.BlockSpec((1, d), full), pl.BlockSpec((d, d), full),
                  pl.BlockSpec((1, d), full)],
        out_specs=pl.BlockSpec((tn, d), blk),
        out_shape=jax.ShapeDtypeStruct((n, d), jnp.float32),
    )(h, hn, sums, cnt, w1h, w1m, bn1.reshape(1, d), wn2, bn2.reshape(1, d))


def kernel(h, rbf_edge, gram_edge, edge_index, ln_g, ln_b,
           We1, be1, We2, be2, Wn1, bn1, Wn2, bn2):
    n, d = h.shape
    src = edge_index[0]
    dst = edge_index[1]
    g = gram_edge.shape[1]

    w_src = We1[:d]
    w_dst = We1[d:2 * d]
    w_g = We1[2 * d:2 * d + g]
    w_r = We1[2 * d + g:]

    hn, a_proj, b_proj = _node_pre(h, ln_g, ln_b, w_src, w_dst, be1, tn=1000)
    gsum = _gather(a_proj, b_proj, src, dst)
    cnt = _scatter_counts(dst, n, d)
    m2 = _edge_mlp(gsum, gram_edge, rbf_edge, w_g, w_r, We2, be2,
                   te=2000)
    sums = _scatter_sums(m2, dst, n)
    return _node_update(h, hn, sums, cnt, Wn1[:d], Wn1[d:], bn1, Wn2, bn2,
                        tn=1000)
